# baseline (speedup 1.0000x reference)
import jax
import jax.numpy as jnp
from jax.experimental import pallas as pl
from jax.experimental.pallas import tpu as pltpu

_B = 1024


def _prep_kernel(y_ref, z_ref, yn_ref, zn_ref, sim_ref, d_ref, cnt_ref):
    bi, bj = sim_ref.shape
    yb = y_ref[...]
    zb = z_ref[...]
    yn = yb * (1.0 / jnp.sqrt(jnp.sum(yb * yb, axis=1, keepdims=True)))
    zn = zb * (1.0 / jnp.sqrt(jnp.sum(zb * zb, axis=1, keepdims=True)))
    ynb = yn.astype(jnp.bfloat16)
    znb = zn.astype(jnp.bfloat16)
    yn_ref[...] = ynb
    zn_ref[...] = znb
    s = jax.lax.dot_general(
        ynb, znb, (((1,), (1,)), ((), ())), preferred_element_type=jnp.float32)
    sim_ref[...] = s
    row_l = jax.lax.broadcasted_iota(jnp.int32, (bi, bj), 0)
    col_l = jax.lax.broadcasted_iota(jnp.int32, (bi, bj), 1)
    d = jnp.sum(jnp.where(row_l == col_l, s, 0.0), axis=1, keepdims=True)
    d_ref[...] = d
    r = jnp.where(s > d, 1.0, 0.0)
    r = r + jnp.where((s == d) & (col_l < row_l), 1.0, 0.0)
    cnt_ref[...] = jnp.sum(r, axis=1, keepdims=True)


def _main_kernel(sim_in, yn_ref, zn_hbm, d_ref, cntin_ref,
                 sim_ref, cnt_ref, zn_vmem, sem):
    del sim_in
    i = pl.program_id(0)
    j = pl.program_id(1)
    nj = pl.num_programs(1) + 1
    bi = sim_ref.shape[0]

    @pl.when((i == 0) & (j == 0))
    def _():
        cp = pltpu.make_async_copy(zn_hbm, zn_vmem, sem)
        cp.start()
        cp.wait()

    j_actual = jax.lax.rem(i + 1 + j, nj)
    s = jax.lax.dot_general(
        yn_ref[...], zn_vmem[pl.ds(j_actual * bi, bi), :],
        (((1,), (1,)), ((), ())),
        preferred_element_type=jnp.float32)
    sim_ref[...] = s
    d = d_ref[pl.ds(i * bi, bi), :]
    tie = jnp.where(j_actual < i, 1.0, 0.0)
    r = jnp.where(s > d, 1.0, 0.0) + jnp.where(s == d, tie, 0.0)
    rsum = jnp.sum(r, axis=1, keepdims=True)

    @pl.when(j == 0)
    def _():
        cnt_ref[...] = cntin_ref[...] + rsum

    @pl.when(j != 0)
    def _():
        cnt_ref[...] += rsum


def _acc_kernel(cnt_ref, t1_ref, t10_ref):
    cnt = cnt_ref[...]
    n = cnt.shape[0]
    t1_ref[0, 0] = jnp.sum((cnt == 0.0).astype(jnp.float32)) * (1.0 / n)
    t10_ref[0, 0] = jnp.sum((cnt < 10.0).astype(jnp.float32)) * (1.0 / n)


def kernel(Z, Y):
    b, f = Z.shape
    nb = b // _B

    yn, zn, sim0, d, cnt0 = pl.pallas_call(
        _prep_kernel,
        grid=(nb,),
        in_specs=[
            pl.BlockSpec((_B, f), lambda k: (k, 0)),
            pl.BlockSpec((_B, f), lambda k: (k, 0)),
        ],
        out_specs=[
            pl.BlockSpec((_B, f), lambda k: (k, 0)),
            pl.BlockSpec((_B, f), lambda k: (k, 0)),
            pl.BlockSpec((_B, _B), lambda k: (k, k)),
            pl.BlockSpec((_B, 1), lambda k: (k, 0)),
            pl.BlockSpec((_B, 1), lambda k: (k, 0)),
        ],
        out_shape=[
            jax.ShapeDtypeStruct((b, f), jnp.bfloat16),
            jax.ShapeDtypeStruct((b, f), jnp.bfloat16),
            jax.ShapeDtypeStruct((b, b), jnp.float32),
            jax.ShapeDtypeStruct((b, 1), jnp.float32),
            jax.ShapeDtypeStruct((b, 1), jnp.float32),
        ],
        compiler_params=pltpu.CompilerParams(
            dimension_semantics=("arbitrary",),
            vmem_limit_bytes=65280 * 1024,
        ),
    )(Y, Z)

    sim, cnt = pl.pallas_call(
        _main_kernel,
        grid=(nb, nb - 1),
        in_specs=[
            pl.BlockSpec(memory_space=pl.ANY),
            pl.BlockSpec((_B, f), lambda i, j: (i, 0)),
            pl.BlockSpec(memory_space=pl.ANY),
            pl.BlockSpec((b, 1), lambda i, j: (0, 0)),
            pl.BlockSpec((_B, 1), lambda i, j: (i, 0)),
        ],
        out_specs=[
            pl.BlockSpec((_B, _B), lambda i, j, nb=nb: (i, (i + 1 + j) % nb)),
            pl.BlockSpec((_B, 1), lambda i, j: (i, 0)),
        ],
        out_shape=[
            jax.ShapeDtypeStruct((b, b), jnp.float32),
            jax.ShapeDtypeStruct((b, 1), jnp.float32),
        ],
        scratch_shapes=[
            pltpu.VMEM((b, f), jnp.bfloat16),
            pltpu.SemaphoreType.DMA,
        ],
        input_output_aliases={0: 0},
        compiler_params=pltpu.CompilerParams(
            dimension_semantics=("arbitrary", "arbitrary"),
            vmem_limit_bytes=62 * 1024 * 1024,
        ),
    )(sim0, yn, zn, d, cnt0)

    t1, t10 = pl.pallas_call(
        _acc_kernel,
        out_specs=[
            pl.BlockSpec(memory_space=pltpu.SMEM),
            pl.BlockSpec(memory_space=pltpu.SMEM),
        ],
        out_shape=[
            jax.ShapeDtypeStruct((1, 1), jnp.float32),
            jax.ShapeDtypeStruct((1, 1), jnp.float32),
        ],
    )(cnt)

    return (t1[0, 0], t10[0, 0], sim)


# speedup vs baseline: 1.0392x; 1.0392x over previous
import jax
import jax.numpy as jnp
from jax.experimental import pallas as pl
from jax.experimental.pallas import tpu as pltpu

_B = 1024


def _prep_kernel(y_ref, z_ref, yn_ref, zn_ref, sim_ref, d_ref, cnt_ref):
    bi, bj = sim_ref.shape
    yb = y_ref[...]
    zb = z_ref[...]
    yn = yb * (1.0 / jnp.sqrt(jnp.sum(yb * yb, axis=1, keepdims=True)))
    zn = zb * (1.0 / jnp.sqrt(jnp.sum(zb * zb, axis=1, keepdims=True)))
    ynb = yn.astype(jnp.bfloat16)
    znb = zn.astype(jnp.bfloat16)
    yn_ref[...] = ynb
    zn_ref[...] = znb
    s = jax.lax.dot_general(
        ynb, znb, (((1,), (1,)), ((), ())), preferred_element_type=jnp.float32)
    sim_ref[...] = s
    row_l = jax.lax.broadcasted_iota(jnp.int32, (bi, bj), 0)
    col_l = jax.lax.broadcasted_iota(jnp.int32, (bi, bj), 1)
    d = jnp.sum(jnp.where(row_l == col_l, s, 0.0), axis=1, keepdims=True)
    d_ref[...] = d
    r = jnp.where(s > d, 1.0, 0.0)
    r = r + jnp.where((s == d) & (col_l < row_l), 1.0, 0.0)
    cnt_ref[...] = jnp.sum(r, axis=1, keepdims=True)


def _main_kernel(sim_in, yn_ref, zn_ref, d_ref, cntin_ref,
                 sim_ref, cnt_ref):
    del sim_in
    i = pl.program_id(0)
    j = pl.program_id(1)
    nj = pl.num_programs(1) + 1
    bi = sim_ref.shape[0]

    j_actual = jax.lax.rem(i + 1 + j, nj)
    s = jax.lax.dot_general(
        yn_ref[...], zn_ref[...], (((1,), (1,)), ((), ())),
        preferred_element_type=jnp.float32)
    sim_ref[...] = s
    d = d_ref[pl.ds(i * bi, bi), :]
    tie = jnp.where(j_actual < i, 1.0, 0.0)
    r = jnp.where(s > d, 1.0, 0.0) + jnp.where(s == d, tie, 0.0)
    rsum = jnp.sum(r, axis=1, keepdims=True)

    @pl.when(j == 0)
    def _():
        cnt_ref[...] = cntin_ref[...] + rsum

    @pl.when(j != 0)
    def _():
        cnt_ref[...] += rsum


def _acc_kernel(cnt_ref, t1_ref, t10_ref):
    cnt = cnt_ref[...]
    n = cnt.shape[0]
    t1_ref[0, 0] = jnp.sum((cnt == 0.0).astype(jnp.float32)) * (1.0 / n)
    t10_ref[0, 0] = jnp.sum((cnt < 10.0).astype(jnp.float32)) * (1.0 / n)


def kernel(Z, Y):
    b, f = Z.shape
    nb = b // _B

    yn, zn, sim0, d, cnt0 = pl.pallas_call(
        _prep_kernel,
        grid=(nb,),
        in_specs=[
            pl.BlockSpec((_B, f), lambda k: (k, 0)),
            pl.BlockSpec((_B, f), lambda k: (k, 0)),
        ],
        out_specs=[
            pl.BlockSpec((_B, f), lambda k: (k, 0)),
            pl.BlockSpec((_B, f), lambda k: (k, 0)),
            pl.BlockSpec((_B, _B), lambda k: (k, k)),
            pl.BlockSpec((_B, 1), lambda k: (k, 0)),
            pl.BlockSpec((_B, 1), lambda k: (k, 0)),
        ],
        out_shape=[
            jax.ShapeDtypeStruct((b, f), jnp.bfloat16),
            jax.ShapeDtypeStruct((b, f), jnp.bfloat16),
            jax.ShapeDtypeStruct((b, b), jnp.float32),
            jax.ShapeDtypeStruct((b, 1), jnp.float32),
            jax.ShapeDtypeStruct((b, 1), jnp.float32),
        ],
        compiler_params=pltpu.CompilerParams(
            dimension_semantics=("arbitrary",),
            vmem_limit_bytes=65280 * 1024,
        ),
    )(Y, Z)

    sim, cnt = pl.pallas_call(
        _main_kernel,
        grid=(nb, nb - 1),
        in_specs=[
            pl.BlockSpec(memory_space=pl.ANY),
            pl.BlockSpec((_B, f), lambda i, j: (i, 0)),
            pl.BlockSpec((_B, f), lambda i, j, nb=nb: ((i + 1 + j) % nb, 0)),
            pl.BlockSpec((b, 1), lambda i, j: (0, 0)),
            pl.BlockSpec((_B, 1), lambda i, j: (i, 0)),
        ],
        out_specs=[
            pl.BlockSpec((_B, _B), lambda i, j, nb=nb: (i, (i + 1 + j) % nb)),
            pl.BlockSpec((_B, 1), lambda i, j: (i, 0)),
        ],
        out_shape=[
            jax.ShapeDtypeStruct((b, b), jnp.float32),
            jax.ShapeDtypeStruct((b, 1), jnp.float32),
        ],
        input_output_aliases={0: 0},
        compiler_params=pltpu.CompilerParams(
            dimension_semantics=("arbitrary", "arbitrary"),
            vmem_limit_bytes=62 * 1024 * 1024,
        ),
    )(sim0, yn, zn, d, cnt0)

    t1, t10 = pl.pallas_call(
        _acc_kernel,
        out_specs=[
            pl.BlockSpec(memory_space=pltpu.SMEM),
            pl.BlockSpec(memory_space=pltpu.SMEM),
        ],
        out_shape=[
            jax.ShapeDtypeStruct((1, 1), jnp.float32),
            jax.ShapeDtypeStruct((1, 1), jnp.float32),
        ],
    )(cnt)

    return (t1[0, 0], t10[0, 0], sim)
